# baseline (speedup 1.0000x reference)
import functools

import jax
import jax.numpy as jnp
from jax import lax
from jax.experimental import pallas as pl
from jax.experimental.pallas import tpu as pltpu
from jax.experimental.pallas import tpu_sc as plsc

N_USERS = 1000000
N_ITEMS = 1000000
EMBED_DIM = 64
BATCH = 16384

NC = 2
NS = 16
L = 16
NW = NC * NS
B_PER_W = BATCH // NW
IDX_CHUNK = 128
N_CHUNKS = B_PER_W // IDX_CHUNK
GROUPS = B_PER_W // L


def _mf_kernel(u_hbm, i_hbm, uw_hbm, iw_hbm, out_hbm,
               uidx_v, iidx_v, ue_v, ie_v, out_v, sem):
    wid = lax.axis_index("s") * NC + lax.axis_index("c")
    base = pl.multiple_of(wid * B_PER_W, B_PER_W)

    pltpu.sync_copy(u_hbm.at[pl.ds(base, B_PER_W)], uidx_v)
    pltpu.sync_copy(i_hbm.at[pl.ds(base, B_PER_W)], iidx_v)

    copies = []
    for j in range(N_CHUNKS):
        sl = pl.ds(j * IDX_CHUNK, IDX_CHUNK)
        copies.append(pltpu.async_copy(
            uw_hbm.at[uidx_v.at[sl]], ue_v.at[sl], sem))
        copies.append(pltpu.async_copy(
            iw_hbm.at[iidx_v.at[sl]], ie_v.at[sl], sem))
    for cp in copies:
        cp.wait()

    iota = lax.iota(jnp.int32, L)
    ones = jnp.ones((L,), jnp.int32)

    def group_body(g, _):
        rows = jnp.full((L,), g * L, jnp.int32) + iota
        col = jnp.zeros((L,), jnp.int32)
        accs = [jnp.zeros((L,), jnp.float32) for _ in range(4)]
        for d in range(EMBED_DIM):
            a = plsc.load_gather(ue_v, [rows, col])
            b = plsc.load_gather(ie_v, [rows, col])
            accs[d % 4] = accs[d % 4] + a * b
            if d != EMBED_DIM - 1:
                col = col + ones
        out_v[pl.ds(pl.multiple_of(g * L, L), L)] = (
            (accs[0] + accs[1]) + (accs[2] + accs[3]))
        return _

    lax.fori_loop(0, GROUPS, group_body, 0, unroll=False)

    pltpu.sync_copy(out_v, out_hbm.at[pl.ds(base, B_PER_W)])


@jax.jit
def kernel(u, i, user_w, item_w):
    mesh = plsc.VectorSubcoreMesh(core_axis_name="c", subcore_axis_name="s")
    run = functools.partial(
        pl.kernel, mesh=mesh,
        compiler_params=pltpu.CompilerParams(
            use_tc_tiling_on_sc=False, needs_layout_passes=False),
        out_type=jax.ShapeDtypeStruct((BATCH,), jnp.float32),
        scratch_types=[
            pltpu.VMEM((B_PER_W,), jnp.int32),
            pltpu.VMEM((B_PER_W,), jnp.int32),
            pltpu.VMEM((B_PER_W, EMBED_DIM), jnp.float32),
            pltpu.VMEM((B_PER_W, EMBED_DIM), jnp.float32),
            pltpu.VMEM((B_PER_W,), jnp.float32),
            pltpu.SemaphoreType.DMA,
        ],
    )(_mf_kernel)
    return run(u.astype(jnp.int32), i.astype(jnp.int32), user_w, item_w)
